# Initial kernel scaffold; baseline (speedup 1.0000x reference)
import jax
import jax.numpy as jnp
from jax.experimental import pallas as pl


def kernel(q_len, k_len, relative_attention_bias):
    raise NotImplementedError("write your pallas kernel here")


# speedup vs baseline: 42.1698x; 42.1698x over previous
import functools

import jax
import jax.numpy as jnp
from jax import lax
from jax.experimental import pallas as pl
from jax.experimental.pallas import tpu as pltpu
from jax.experimental.pallas import tpu_sc as plsc

NUM_BUCKETS = 32
NUM_HEADS = 16
Q_LEN = 2048
K_LEN = 2048

NCORES = 2
NSUB = 16

ROWS = 16
S_COLS = 2 * K_LEN + ROWS
Q_PER_WORKER = Q_LEN // 2
TILES = Q_PER_WORKER // ROWS


def _sc_body(table_hbm, out_hbm, table_v, s_v, sem):
    cid = lax.axis_index("c")
    sid = lax.axis_index("s")
    wid = sid * NCORES + cid
    h = wid // 2
    half = wid % 2
    q_base = half * Q_PER_WORKER

    pltpu.sync_copy(table_hbm, table_v)

    lane = lax.iota(jnp.int32, 16)
    h_idx = jnp.full((16,), h, dtype=jnp.int32)

    def scol(c, carry):
        j = c * 16 + lane
        for i in range(8):
            b = jnp.clip(j - (i + 1912), 0, 255) & (NUM_BUCKETS - 1)
            s_v[i, pl.ds(c * 16, 16)] = plsc.load_gather(table_v, [b, h_idx])
        return carry

    lax.fori_loop(0, S_COLS // 16, scol, 0)

    def scol2(c, carry):
        for i in range(8):
            s_v[i + 8, pl.ds(c * 16 - 8, 16)] = s_v[i, pl.ds(c * 16 - 16, 16)]
        return carry

    lax.fori_loop(1, S_COLS // 16, scol2, 0)

    def mk(t):
        q0 = q_base + t * ROWS
        j0 = (Q_LEN - 8) - q0
        return pltpu.make_async_copy(
            s_v.at[:, pl.ds(j0, K_LEN)],
            out_hbm.at[pl.ds(h * Q_LEN + q0, ROWS), :],
            sem,
        )

    mk(0).start()

    def step(t, carry):
        mk(t).start()
        mk(t - 1).wait()
        return carry

    lax.fori_loop(1, TILES, step, 0)
    mk(TILES - 1).wait()


def kernel(q_len, k_len, relative_attention_bias):
    mesh = plsc.VectorSubcoreMesh(
        core_axis_name="c", subcore_axis_name="s",
        num_cores=NCORES, num_subcores=NSUB,
    )
    run = functools.partial(
        pl.kernel,
        out_type=jax.ShapeDtypeStruct((NUM_HEADS * Q_LEN, K_LEN), jnp.float32),
        mesh=mesh,
        scratch_types=[
            pltpu.VMEM((NUM_BUCKETS, NUM_HEADS), jnp.float32),
            pltpu.VMEM((ROWS, S_COLS), jnp.float32),
            pltpu.SemaphoreType.DMA,
        ],
        compiler_params=pltpu.CompilerParams(
            use_tc_tiling_on_sc=False, needs_layout_passes=False,
        ),
    )(_sc_body)
    out = run(relative_attention_bias)
    return out.reshape(1, NUM_HEADS, Q_LEN, K_LEN)


# speedup vs baseline: 42.6961x; 1.0125x over previous
import functools

import jax
import jax.numpy as jnp
from jax import lax
from jax.experimental import pallas as pl
from jax.experimental.pallas import tpu as pltpu
from jax.experimental.pallas import tpu_sc as plsc

NUM_BUCKETS = 32
NUM_HEADS = 16
Q_LEN = 2048
K_LEN = 2048

NCORES = 2
NSUB = 16

ROWS = 16
S_COLS = 2 * K_LEN + ROWS
Q_PER_WORKER = Q_LEN // 2
TILES = Q_PER_WORKER // ROWS


def _sc_body(table_hbm, out_hbm, table_v, s_v, sem):
    cid = lax.axis_index("c")
    sid = lax.axis_index("s")
    wid = sid * NCORES + cid
    h = wid // 2
    half = wid % 2
    q_base = half * Q_PER_WORKER

    pltpu.sync_copy(table_hbm, table_v)

    lane = lax.iota(jnp.int32, 16)
    h_idx = jnp.full((16,), h, dtype=jnp.int32)

    def scol(c, carry):
        j = c * 16 + lane
        for i in range(8):
            b = jnp.clip(j - (i + 1912), 0, 255) & (NUM_BUCKETS - 1)
            s_v[i, pl.ds(c * 16, 16)] = plsc.load_gather(table_v, [b, h_idx])
        return carry

    lax.fori_loop(0, S_COLS // 16, scol, 0)

    def scol2(c, carry):
        for i in range(8):
            s_v[i + 8, pl.ds(c * 16 - 8, 16)] = s_v[i, pl.ds(c * 16 - 16, 16)]
        return carry

    lax.fori_loop(1, S_COLS // 16, scol2, 0)

    def mk(t):
        q0 = q_base + t * ROWS
        j0 = (Q_LEN - 8) - q0
        return pltpu.make_async_copy(
            s_v.at[:, pl.ds(j0, K_LEN)],
            out_hbm.at[0, h, pl.ds(q0, ROWS), :],
            sem,
        )

    mk(0).start()

    def step(t, carry):
        mk(t).start()
        mk(t - 1).wait()
        return carry

    lax.fori_loop(1, TILES, step, 0)
    mk(TILES - 1).wait()


def kernel(q_len, k_len, relative_attention_bias):
    mesh = plsc.VectorSubcoreMesh(
        core_axis_name="c", subcore_axis_name="s",
        num_cores=NCORES, num_subcores=NSUB,
    )
    run = functools.partial(
        pl.kernel,
        out_type=jax.ShapeDtypeStruct((1, NUM_HEADS, Q_LEN, K_LEN), jnp.float32),
        mesh=mesh,
        scratch_types=[
            pltpu.VMEM((NUM_BUCKETS, NUM_HEADS), jnp.float32),
            pltpu.VMEM((ROWS, S_COLS), jnp.float32),
            pltpu.SemaphoreType.DMA,
        ],
        compiler_params=pltpu.CompilerParams(
            use_tc_tiling_on_sc=False, needs_layout_passes=False,
        ),
    )(_sc_body)
    return run(relative_attention_bias)


# speedup vs baseline: 99.8411x; 2.3384x over previous
import functools

import jax
import jax.numpy as jnp
from jax import lax
from jax.experimental import pallas as pl
from jax.experimental.pallas import tpu as pltpu
from jax.experimental.pallas import tpu_sc as plsc

NUM_BUCKETS = 32
NUM_HEADS = 16
Q_LEN = 2048
K_LEN = 2048

NCORES = 2
NSUB = 16
HEADS_PER_SC = NUM_HEADS // NCORES

ROWS = 128
S_COLS = 3968
ROWS_PER_TILE = ROWS // NSUB
GROUPS = S_COLS // 128
BAND_GROUPS = 3


def _sc_body(table_hbm, out_hbm, table_v, b2, s2, sem0, sem1):
    cid = lax.axis_index("c")
    t = lax.axis_index("s")

    pltpu.sync_copy(table_hbm, table_v)

    lane = lax.iota(jnp.int32, 16)
    h_base = cid * HEADS_PER_SC
    sems = (sem0, sem1)

    def scatter(hh):
        q0 = pl.multiple_of(t * ROWS, 128)
        j0 = pl.multiple_of((Q_LEN - ROWS) - q0, 128)
        return pltpu.make_async_copy(
            s2.at[hh % 2, :, pl.ds(j0, K_LEN)],
            out_hbm.at[0, h_base + hh, pl.ds(q0, ROWS), :],
            sems[hh % 2],
        )

    for hh in range(HEADS_PER_SC):
        h = h_base + hh

        c0 = plsc.load_gather(table_v, [jnp.full((16,), h, jnp.int32)])
        c31 = plsc.load_gather(
            table_v, [jnp.full((16,), (NUM_BUCKETS - 1) * NUM_HEADS + h,
                               jnp.int32)])

        def build_row(i, carry):
            r_abs = t * ROWS_PER_TILE + i
            g_lo = (1793 + r_abs) >> 7
            x_off = 127 - r_abs - 1919

            def fill(g, fc):
                val = jnp.where(g < g_lo, c0, c31)
                for u in range(8):
                    b2[i, pl.ds(g * 128 + u * 16, 16)] = val
                return fc

            lax.fori_loop(0, GROUPS, fill, 0)

            for g2 in range(BAND_GROUPS):
                for u in range(8):
                    col = (g_lo + g2) * 128 + u * 16
                    b = jnp.clip(col + lane + x_off, 0, 255) & (NUM_BUCKETS - 1)
                    b2[i, pl.ds(col, 16)] = plsc.load_gather(
                        table_v, [b * NUM_HEADS + h])
            return carry

        lax.fori_loop(0, ROWS_PER_TILE, build_row, 0)

        if hh >= 2:
            scatter(hh - 2).wait()
        plsc.subcore_barrier()
        row0 = pl.multiple_of(t * ROWS_PER_TILE, 8)
        pltpu.sync_copy(b2, s2.at[hh % 2, pl.ds(row0, ROWS_PER_TILE), :])
        plsc.subcore_barrier()
        scatter(hh).start()

    scatter(HEADS_PER_SC - 2).wait()
    scatter(HEADS_PER_SC - 1).wait()


def kernel(q_len, k_len, relative_attention_bias):
    mesh = plsc.VectorSubcoreMesh(
        core_axis_name="c", subcore_axis_name="s",
        num_cores=NCORES, num_subcores=NSUB,
    )
    run = functools.partial(
        pl.kernel,
        out_type=jax.ShapeDtypeStruct((1, NUM_HEADS, Q_LEN, K_LEN), jnp.float32),
        mesh=mesh,
        scratch_types=[
            pltpu.VMEM((NUM_BUCKETS * NUM_HEADS,), jnp.float32),
            pltpu.VMEM((ROWS_PER_TILE, S_COLS), jnp.float32),
            pltpu.VMEM_SHARED((2, ROWS, S_COLS), jnp.float32),
            pltpu.SemaphoreType.DMA,
            pltpu.SemaphoreType.DMA,
        ],
        compiler_params=pltpu.CompilerParams(
            use_tc_tiling_on_sc=True, needs_layout_passes=False,
        ),
    )(_sc_body)
    return run(relative_attention_bias.reshape(-1))


# speedup vs baseline: 131.7396x; 1.3195x over previous
import functools

import jax
import jax.numpy as jnp
from jax import lax
from jax.experimental import pallas as pl
from jax.experimental.pallas import tpu as pltpu
from jax.experimental.pallas import tpu_sc as plsc

NUM_BUCKETS = 32
NUM_HEADS = 16
Q_LEN = 2048
K_LEN = 2048

NCORES = 2
NSUB = 16
HEADS_PER_SC = NUM_HEADS // NCORES

B = 128
NBLK = Q_LEN // B


def _sc_body(table_hbm, out_hbm, table_v, pan, sem0, sem1):
    cid = lax.axis_index("c")
    t = lax.axis_index("s")

    pltpu.sync_copy(table_hbm, table_v)

    lane = lax.iota(jnp.int32, 16)
    h_base = cid * HEADS_PER_SC
    sems = (sem0, sem1)
    panels = (t, t + NBLK)

    def build_panel(h, p, dst):
        c0 = plsc.load_gather(table_v, [jnp.full((16,), h, jnp.int32)])
        c31 = plsc.load_gather(
            table_v, [jnp.full((16,), (NUM_BUCKETS - 1) * NUM_HEADS + h,
                               jnp.int32)])
        lo = jnp.clip(B * p - 2047, 0, B)
        hi = jnp.clip(B * p - 1664, 0, B)
        base = B * p + 127 - 1919

        def const_row(val):
            def body(i, carry):
                for u in range(8):
                    dst[i, pl.ds(u * 16, 16)] = val
                return carry
            return body

        def band_row(i, carry):
            off = base - i
            for u in range(8):
                b = jnp.clip(u * 16 + lane + off, 0, 255) & (NUM_BUCKETS - 1)
                dst[i, pl.ds(u * 16, 16)] = plsc.load_gather(
                    table_v, [b * NUM_HEADS + h])
            return carry

        lax.fori_loop(0, lo, const_row(c31), 0)
        lax.fori_loop(lo, hi, band_row, 0)
        lax.fori_loop(hi, B, const_row(c0), 0)

    def scatter(hh):
        h = h_base + hh
        buf = hh % 2
        sem = sems[buf]

        def blk(a, p_k):
            tp = p_k + 15 - panels[a]
            q0 = pl.multiple_of(tp * B, 128)
            k0 = pl.multiple_of(p_k * B, 128)
            return pltpu.make_async_copy(
                pan.at[buf, a],
                out_hbm.at[0, h, pl.ds(q0, B), pl.ds(k0, B)],
                sem,
            )
        return blk

    def fire(hh):
        blk = scatter(hh)
        lax.fori_loop(0, t + 1, lambda k, c: (blk(0, k).start(), c)[1], 0)
        lax.fori_loop(t + 1, NBLK, lambda k, c: (blk(1, k).start(), c)[1], 0)

    def drain(hh):
        blk = scatter(hh)
        lax.fori_loop(0, t + 1, lambda k, c: (blk(0, k).wait(), c)[1], 0)
        lax.fori_loop(t + 1, NBLK, lambda k, c: (blk(1, k).wait(), c)[1], 0)

    for hh in range(HEADS_PER_SC):
        if hh >= 2:
            drain(hh - 2)
        h = h_base + hh
        build_panel(h, panels[0], pan.at[hh % 2, 0])
        build_panel(h, panels[1], pan.at[hh % 2, 1])
        fire(hh)

    drain(HEADS_PER_SC - 2)
    drain(HEADS_PER_SC - 1)


def kernel(q_len, k_len, relative_attention_bias):
    mesh = plsc.VectorSubcoreMesh(
        core_axis_name="c", subcore_axis_name="s",
        num_cores=NCORES, num_subcores=NSUB,
    )
    run = functools.partial(
        pl.kernel,
        out_type=jax.ShapeDtypeStruct((1, NUM_HEADS, Q_LEN, K_LEN), jnp.float32),
        mesh=mesh,
        scratch_types=[
            pltpu.VMEM((NUM_BUCKETS * NUM_HEADS,), jnp.float32),
            pltpu.VMEM((2, 2, B, B), jnp.float32),
            pltpu.SemaphoreType.DMA,
            pltpu.SemaphoreType.DMA,
        ],
        compiler_params=pltpu.CompilerParams(
            use_tc_tiling_on_sc=True, needs_layout_passes=False,
        ),
    )(_sc_body)
    return run(relative_attention_bias.reshape(-1))


# speedup vs baseline: 135.7352x; 1.0303x over previous
import functools

import jax
import jax.numpy as jnp
from jax import lax
from jax.experimental import pallas as pl
from jax.experimental.pallas import tpu as pltpu
from jax.experimental.pallas import tpu_sc as plsc

NUM_BUCKETS = 32
NUM_HEADS = 16
Q_LEN = 2048
K_LEN = 2048

NCORES = 2
NSUB = 16
HEADS_PER_SC = NUM_HEADS // NCORES

B = 128
NBLK = Q_LEN // B


def _sc_body(table_hbm, out_hbm, table_v, pan, sem0, sem1):
    cid = lax.axis_index("c")
    t = lax.axis_index("s")

    pltpu.sync_copy(table_hbm, table_v)

    lane = lax.iota(jnp.int32, 16)
    h_base = cid * HEADS_PER_SC
    sems = (sem0, sem1)
    panels = (t, t + NBLK)

    def build_panel(h, p, dst):
        c0 = plsc.load_gather(table_v, [jnp.full((16,), h, jnp.int32)])
        c31 = plsc.load_gather(
            table_v, [jnp.full((16,), (NUM_BUCKETS - 1) * NUM_HEADS + h,
                               jnp.int32)])
        lo = jnp.clip(B * p - 2047, 0, B)
        hi = jnp.clip(B * p - 1664, 0, B)
        base = B * p + 127 - 1919

        def const_row(val):
            def body(i, carry):
                for u in range(8):
                    dst[i, pl.ds(u * 16, 16)] = val
                return carry
            return body

        def band_row(i, carry):
            off = base - i
            for u in range(8):
                b = jnp.clip(u * 16 + lane + off, 0, 255) & (NUM_BUCKETS - 1)
                dst[i, pl.ds(u * 16, 16)] = plsc.load_gather(
                    table_v, [b * NUM_HEADS + h])
            return carry

        lax.fori_loop(0, lo, const_row(c31), 0)
        lax.fori_loop(lo, hi, band_row, 0)
        lax.fori_loop(hi, B, const_row(c0), 0)

    def blocks(h, buf):
        def blk(a, p_k):
            tp = p_k + 15 - panels[a]
            q0 = pl.multiple_of(tp * B, 128)
            k0 = pl.multiple_of(p_k * B, 128)
            return pltpu.make_async_copy(
                pan.at[buf, a],
                out_hbm.at[0, h, pl.ds(q0, B), pl.ds(k0, B)],
                sems[buf],
            )
        return blk

    def fire(h, buf):
        blk = blocks(h, buf)
        lax.fori_loop(0, t + 1, lambda k, c: (blk(0, k).start(), c)[1], 0)
        lax.fori_loop(t + 1, NBLK, lambda k, c: (blk(1, k).start(), c)[1], 0)

    def drain(buf):
        blk = blocks(h_base, buf)
        lax.fori_loop(0, t + 1, lambda k, c: (blk(0, k).wait(), c)[1], 0)
        lax.fori_loop(t + 1, NBLK, lambda k, c: (blk(1, k).wait(), c)[1], 0)

    def head(hh, buf):
        h = h_base + hh
        build_panel(h, panels[0], pan.at[buf, 0])
        build_panel(h, panels[1], pan.at[buf, 1])
        fire(h, buf)

    head(0, 0)
    head(1, 1)

    def pair(it, carry):
        for par in range(2):
            drain(par)
            head(2 * it + par, par)
        return carry

    lax.fori_loop(1, HEADS_PER_SC // 2, pair, 0)
    drain(0)
    drain(1)


def kernel(q_len, k_len, relative_attention_bias):
    mesh = plsc.VectorSubcoreMesh(
        core_axis_name="c", subcore_axis_name="s",
        num_cores=NCORES, num_subcores=NSUB,
    )
    run = functools.partial(
        pl.kernel,
        out_type=jax.ShapeDtypeStruct((1, NUM_HEADS, Q_LEN, K_LEN), jnp.float32),
        mesh=mesh,
        scratch_types=[
            pltpu.VMEM((NUM_BUCKETS * NUM_HEADS,), jnp.float32),
            pltpu.VMEM((2, 2, B, B), jnp.float32),
            pltpu.SemaphoreType.DMA,
            pltpu.SemaphoreType.DMA,
        ],
        compiler_params=pltpu.CompilerParams(
            use_tc_tiling_on_sc=True, needs_layout_passes=False,
        ),
    )(_sc_body)
    return run(relative_attention_bias.reshape(-1))


# speedup vs baseline: 137.8850x; 1.0158x over previous
import functools

import jax
import jax.numpy as jnp
from jax import lax
from jax.experimental import pallas as pl
from jax.experimental.pallas import tpu as pltpu
from jax.experimental.pallas import tpu_sc as plsc

NUM_BUCKETS = 32
NUM_HEADS = 16
Q_LEN = 2048
K_LEN = 2048

NCORES = 2
NSUB = 16
HEADS_PER_SC = NUM_HEADS // NCORES

B = 128
NBLK = Q_LEN // B


def _sc_body(table_hbm, out_hbm, table_v, pan, sem0, sem1):
    cid = lax.axis_index("c")
    t = lax.axis_index("s")

    pltpu.sync_copy(table_hbm, table_v)

    lane = lax.iota(jnp.int32, 16)
    h_base = cid * HEADS_PER_SC
    sems = (sem0, sem1)
    panels = (t, t + NBLK)

    def build_panel(h, p, dst):
        c0 = plsc.load_gather(table_v, [jnp.full((16,), h, jnp.int32)])
        c31 = plsc.load_gather(
            table_v, [jnp.full((16,), (NUM_BUCKETS - 1) * NUM_HEADS + h,
                               jnp.int32)])
        lo = jnp.clip(B * p - 2047, 0, B)
        hi = jnp.clip(B * p - 1664, 0, B)
        base = B * p + 127 - 1919

        def const_row(val):
            def body(i, carry):
                for u in range(8):
                    dst[i, pl.ds(u * 16, 16)] = val
                return carry
            return body

        def band_row(i, carry):
            off = base - i
            for u in range(8):
                b = jnp.clip(u * 16 + lane + off, 0, 255) & (NUM_BUCKETS - 1)
                dst[i, pl.ds(u * 16, 16)] = plsc.load_gather(
                    table_v, [b * NUM_HEADS + h])
            return carry

        lax.fori_loop(0, lo, const_row(c31), 0)
        lax.fori_loop(lo, hi, band_row, 0)
        lax.fori_loop(hi, B, const_row(c0), 0)

    def blocks(h, buf):
        def blk(a, p_k):
            tp = p_k + 15 - panels[a]
            q0 = pl.multiple_of(tp * B, 128)
            k0 = pl.multiple_of(p_k * B, 128)
            return pltpu.make_async_copy(
                pan.at[buf, a],
                out_hbm.at[0, h, pl.ds(q0, B), pl.ds(k0, B)],
                sems[buf],
            )
        return blk

    def fire(h, buf):
        blk = blocks(h, buf)
        lax.fori_loop(0, t + 1, lambda k, c: (blk(0, k).start(), c)[1], 0)
        lax.fori_loop(t + 1, NBLK, lambda k, c: (blk(1, k).start(), c)[1], 0)

    def drain(buf):
        blk = blocks(h_base, buf)
        lax.fori_loop(0, t + 1, lambda k, c: (blk(0, k).wait(), c)[1], 0)
        lax.fori_loop(t + 1, NBLK, lambda k, c: (blk(1, k).wait(), c)[1], 0)

    def head(hh, buf):
        h = h_base + hh

        def one_panel(a, carry):
            build_panel(h, t + a * NBLK, pan.at[buf, a])
            return carry

        lax.fori_loop(0, 2, one_panel, 0)
        fire(h, buf)

    def pair(it, carry):
        for par in range(2):

            @pl.when(it > 0)
            def _():
                drain(par)

            head(2 * it + par, par)
        return carry

    lax.fori_loop(0, HEADS_PER_SC // 2, pair, 0)
    drain(0)
    drain(1)


def kernel(q_len, k_len, relative_attention_bias):
    mesh = plsc.VectorSubcoreMesh(
        core_axis_name="c", subcore_axis_name="s",
        num_cores=NCORES, num_subcores=NSUB,
    )
    run = functools.partial(
        pl.kernel,
        out_type=jax.ShapeDtypeStruct((1, NUM_HEADS, Q_LEN, K_LEN), jnp.float32),
        mesh=mesh,
        scratch_types=[
            pltpu.VMEM((NUM_BUCKETS * NUM_HEADS,), jnp.float32),
            pltpu.VMEM((2, 2, B, B), jnp.float32),
            pltpu.SemaphoreType.DMA,
            pltpu.SemaphoreType.DMA,
        ],
        compiler_params=pltpu.CompilerParams(
            use_tc_tiling_on_sc=True, needs_layout_passes=False,
        ),
    )(_sc_body)
    return run(relative_attention_bias.reshape(-1))


# speedup vs baseline: 138.5693x; 1.0050x over previous
import functools

import jax
import jax.numpy as jnp
from jax import lax
from jax.experimental import pallas as pl
from jax.experimental.pallas import tpu as pltpu
from jax.experimental.pallas import tpu_sc as plsc

NUM_BUCKETS = 32
NUM_HEADS = 16
Q_LEN = 2048
K_LEN = 2048

NCORES = 2
NSUB = 16
HEADS_PER_SC = NUM_HEADS // NCORES

B = 128
NBLK = Q_LEN // B


def _sc_body(table_hbm, out_hbm, table_v, pan, sem0, sem1, sem2, sem3):
    cid = lax.axis_index("c")
    t = lax.axis_index("s")

    pltpu.sync_copy(table_hbm, table_v)

    lane = lax.iota(jnp.int32, 16)
    h_base = cid * HEADS_PER_SC
    sems = ((sem0, sem1), (sem2, sem3))
    panels = (t, t + NBLK)

    def build_panel(h, p, dst):
        c0 = plsc.load_gather(table_v, [jnp.full((16,), h, jnp.int32)])
        c31 = plsc.load_gather(
            table_v, [jnp.full((16,), (NUM_BUCKETS - 1) * NUM_HEADS + h,
                               jnp.int32)])
        lo = jnp.clip(B * p - 2047, 0, B)
        hi = jnp.clip(B * p - 1664, 0, B)
        base = B * p + 127 - 1919

        def const_row(val):
            def body(i, carry):
                for u in range(8):
                    dst[i, pl.ds(u * 16, 16)] = val
                return carry
            return body

        def band_row(i, carry):
            off = base - i
            for u in range(8):
                b = jnp.clip(u * 16 + lane + off, 0, 255) & (NUM_BUCKETS - 1)
                dst[i, pl.ds(u * 16, 16)] = plsc.load_gather(
                    table_v, [b * NUM_HEADS + h])
            return carry

        lax.fori_loop(0, lo, const_row(c31), 0)
        lax.fori_loop(lo, hi, band_row, 0)
        lax.fori_loop(hi, B, const_row(c0), 0)

    def blocks(h, buf):
        def blk(a, p_k):
            tp = p_k + 15 - panels[a]
            q0 = pl.multiple_of(tp * B, 128)
            k0 = pl.multiple_of(p_k * B, 128)
            return pltpu.make_async_copy(
                pan.at[buf, a],
                out_hbm.at[0, h, pl.ds(q0, B), pl.ds(k0, B)],
                sems[buf][a],
            )
        return blk

    def fire(h, buf):
        blk = blocks(h, buf)
        lax.fori_loop(0, t + 1, lambda k, c: (blk(0, k).start(), c)[1], 0)
        lax.fori_loop(t + 1, NBLK, lambda k, c: (blk(1, k).start(), c)[1], 0)

    def drain(buf):
        blk = blocks(h_base, buf)
        lax.fori_loop(0, t + 1, lambda k, c: (blk(0, k).wait(), c)[1], 0)
        lax.fori_loop(t + 1, NBLK, lambda k, c: (blk(1, k).wait(), c)[1], 0)

    def head(hh, buf):
        h = h_base + hh

        def one_panel(a, carry):
            build_panel(h, t + a * NBLK, pan.at[buf, a])
            return carry

        lax.fori_loop(0, 2, one_panel, 0)
        fire(h, buf)

    def pair(it, carry):
        for par in range(2):

            @pl.when(it > 0)
            def _():
                drain(par)

            head(2 * it + par, par)
        return carry

    lax.fori_loop(0, HEADS_PER_SC // 2, pair, 0)
    drain(0)
    drain(1)


def kernel(q_len, k_len, relative_attention_bias):
    mesh = plsc.VectorSubcoreMesh(
        core_axis_name="c", subcore_axis_name="s",
        num_cores=NCORES, num_subcores=NSUB,
    )
    run = functools.partial(
        pl.kernel,
        out_type=jax.ShapeDtypeStruct((1, NUM_HEADS, Q_LEN, K_LEN), jnp.float32),
        mesh=mesh,
        scratch_types=[
            pltpu.VMEM((NUM_BUCKETS * NUM_HEADS,), jnp.float32),
            pltpu.VMEM((2, 2, B, B), jnp.float32),
            pltpu.SemaphoreType.DMA,
            pltpu.SemaphoreType.DMA,
            pltpu.SemaphoreType.DMA,
            pltpu.SemaphoreType.DMA,
        ],
        compiler_params=pltpu.CompilerParams(
            use_tc_tiling_on_sc=True, needs_layout_passes=False,
        ),
    )(_sc_body)
    return run(relative_attention_bias.reshape(-1))
